# baseline (speedup 1.0000x reference)
import functools

import jax
import jax.numpy as jnp
from jax import lax
from jax.experimental import pallas as pl
from jax.experimental.pallas import tpu as pltpu
from jax.experimental.pallas import tpu_sc as plsc

EPS_BN = 1e-3
LANES = 16


def _top3_body(xyz1_ref, xyz2t_ref, iw_ref, *, n2):
    b = pl.program_id(0)
    x1 = xyz1_ref[0]
    x2t = xyz2t_ref[0]
    ab = lax.dot_general(x1, x2t, (((1,), (0,)), ((), ())),
                         preferred_element_type=jnp.float32)
    a2 = jnp.sum(x1 * x1, axis=1, keepdims=True)
    b2 = jnp.sum(x2t * x2t, axis=0, keepdims=True)
    d2 = jnp.clip(a2 - 2.0 * ab + b2, 0.0, 1e37)

    t1 = d2.shape[0]
    iota = lax.broadcasted_iota(jnp.int32, (t1, n2), 1)
    bias = jnp.int32(1 << 23)
    keyi = ((lax.bitcast_convert_type(d2, jnp.int32) & ~(n2 - 1)) | iota)
    keyf = lax.bitcast_convert_type(keyi + bias, jnp.float32)
    inf = jnp.float32(jnp.inf)
    ms, idxs = [], []
    for _ in range(3):
        kmf = jnp.min(keyf, axis=1, keepdims=True)
        keyf = jnp.where(keyf == kmf, inf, keyf)
        kmi = lax.bitcast_convert_type(kmf, jnp.int32) - bias
        idxs.append(kmi & (n2 - 1))
        ms.append(lax.bitcast_convert_type(kmi & ~(n2 - 1), jnp.float32))

    inv = [1.0 / (m + 1e-10) for m in ms]
    s = inv[0] + inv[1] + inv[2]
    ws = [v / s for v in inv]

    base = b * n2
    for k in range(3):
        iw_ref[k, 0] = idxs[k] + base
        iw_ref[3 + k, 0] = lax.bitcast_convert_type(ws[k], jnp.int32)


def _top3_call(xyz1, xyz2t, t1=512):
    bsz, n1, _ = xyz1.shape
    n2 = xyz2t.shape[2]
    grid = (bsz, n1 // t1)
    return pl.pallas_call(
        functools.partial(_top3_body, n2=n2),
        grid=grid,
        in_specs=[
            pl.BlockSpec((1, t1, 3), lambda b, i: (b, i, 0)),
            pl.BlockSpec((1, 3, n2), lambda b, i: (b, 0, 0)),
        ],
        out_specs=pl.BlockSpec((6, 1, t1, 1), lambda b, i: (0, b, i, 0)),
        out_shape=jax.ShapeDtypeStruct((6, bsz, n1, 1), jnp.int32),
    )(xyz1, xyz2t)


def _sc_interpolate(iw, feat2f, *, q=32):
    m, cw = iw.shape[1], feat2f.shape[1]
    info = plsc.get_sparse_core_info()
    nw = info.num_cores * info.num_subcores
    nq = m // nw
    nchunk = nq // q
    mesh = plsc.VectorSubcoreMesh(core_axis_name="c", subcore_axis_name="s")
    rows = lambda: pltpu.VMEM((q, cw), jnp.int32)

    @functools.partial(
        pl.kernel, mesh=mesh,
        out_type=jax.ShapeDtypeStruct((m, cw), jnp.int32),
        scratch_types=[
            pltpu.VMEM((nq,), jnp.int32),
            pltpu.VMEM((nq,), jnp.int32),
            pltpu.VMEM((nq,), jnp.int32),
            pltpu.VMEM((nq,), jnp.int32),
            pltpu.VMEM((nq,), jnp.int32),
            pltpu.VMEM((nq,), jnp.int32),
            rows(), rows(), rows(), rows(), rows(), rows(),
            rows(), rows(),
            pltpu.SemaphoreType.DMA, pltpu.SemaphoreType.DMA,
            pltpu.SemaphoreType.DMA, pltpu.SemaphoreType.DMA,
        ],
    )
    def body(iw_h, f2_h, out_h,
             i0_v, i1_v, i2_v, w0_v, w1_v, w2_v,
             r00, r10, r20, r01, r11, r21, o0, o1, gs0, gs1, os0, os1):
        wid = lax.axis_index("s") * info.num_cores + lax.axis_index("c")
        base0 = wid * nq
        sla = pl.ds(base0, nq)
        for k, dst in enumerate((i0_v, i1_v, i2_v, w0_v, w1_v, w2_v)):
            pltpu.sync_copy(iw_h.at[k, sla], dst)

        rbufs = ((r00, r10, r20), (r01, r11, r21))
        obufs = (o0, o1)
        gsems = (gs0, gs1)
        osems = (os0, os1)
        ivs = (i0_v, i1_v, i2_v)
        wvs = (w0_v, w1_v, w2_v)

        def issue(c, p):
            lsl = pl.ds(c * q, q)
            for k in range(3):
                pltpu.async_copy(f2_h.at[ivs[k].at[lsl]], rbufs[p][k],
                                 gsems[p])

        issue(0, 0)

        def pair(cp, _):
            for p in (0, 1):
                c = cp * 2 + p

                @pl.when(c + 1 < nchunk)
                def _prefetch():
                    issue(c + 1, 1 - p)

                for k in range(3):
                    pltpu.make_async_copy(
                        f2_h.at[ivs[k].at[pl.ds(0, q)]], rbufs[p][k],
                        gsems[p]).wait()

                @pl.when(c >= 2)
                def _drain_out():
                    pltpu.make_async_copy(
                        obufs[p], out_h.at[pl.ds(base0, q)], osems[p]).wait()

                wvecs = [[lax.bitcast_convert_type(
                              wv[pl.ds(c * q + h * LANES, LANES)], jnp.float32)
                          for h in range(q // LANES)] for wv in wvs]
                dnums = lax.GatherDimensionNumbers(
                    offset_dims=(), collapsed_slice_dims=(0,),
                    start_index_map=(0,))
                bcast = lambda v, lane: lax.gather(
                    v, lane[:, None], dnums, (1,),
                    mode=lax.GatherScatterMode.PROMISE_IN_BOUNDS)
                for qq in range(q):
                    lane = jnp.full((LANES,), qq % LANES, jnp.int32)
                    h = qq // LANES
                    wa = bcast(wvecs[0][h], lane)
                    wb = bcast(wvecs[1][h], lane)
                    wc = bcast(wvecs[2][h], lane)
                    himask = jnp.int32(-65536)
                    rnd = jnp.int32(0x8000)
                    for j in range(cw // LANES):
                        ds = pl.ds(j * LANES, LANES)
                        ai, bi = [], []
                        for k in range(3):
                            xi = rbufs[p][k][qq, ds]
                            ai.append(lax.bitcast_convert_type(xi << 16, jnp.float32))
                            bi.append(lax.bitcast_convert_type(xi & himask, jnp.float32))
                        oa = ai[0] * wa + ai[1] * wb + ai[2] * wc
                        ob = bi[0] * wa + bi[1] * wb + bi[2] * wc
                        oi = lax.shift_right_logical(
                            lax.bitcast_convert_type(oa, jnp.int32) + rnd, 16) | (
                            (lax.bitcast_convert_type(ob, jnp.int32) + rnd) & himask)
                        obufs[p][qq, ds] = oi
                pltpu.async_copy(obufs[p], out_h.at[pl.ds(base0 + c * q, q)],
                                 osems[p])
            return 0

        lax.fori_loop(0, nchunk // 2, pair, 0)
        for p in (0, 1):
            pltpu.make_async_copy(obufs[p], out_h.at[pl.ds(base0, q)],
                                  osems[p]).wait()

    return body(iw, feat2f)


def _pack_body(x_ref, out_ref):
    xi = lax.bitcast_convert_type(x_ref[...], jnp.int32)
    r = xi + 0x7FFF + (lax.shift_right_logical(xi, 16) & 1)
    n = out_ref.shape[1]
    lo = lax.shift_right_logical(r[:, :n], 16)
    hi = r[:, n:] & jnp.int32(-65536)
    out_ref[...] = lo | hi


def _pack_feat2_call(feat2f, blk=1024):
    n, c2 = feat2f.shape
    return pl.pallas_call(
        _pack_body,
        grid=(n // blk,),
        in_specs=[pl.BlockSpec((blk, c2), lambda i: (i, 0))],
        out_specs=pl.BlockSpec((blk, c2 // 2), lambda i: (i, 0)),
        out_shape=jax.ShapeDtypeStruct((n, c2 // 2), jnp.int32),
    )(feat2f)


def _mlp_body(interp_ref, feat1_ref, w0a1_ref, w0a2_ref, w0b_ref, w1_ref,
              b0_ref, s0_ref, t0_ref, b1_ref, s1_ref, t1_ref, out_ref):
    wi = interp_ref[...]
    lo = lax.bitcast_convert_type(wi << 16, jnp.float32).astype(jnp.bfloat16)
    hi = lax.bitcast_convert_type(wi & jnp.int32(-65536),
                                  jnp.float32).astype(jnp.bfloat16)
    f1 = feat1_ref[...].astype(jnp.bfloat16)
    bf = jnp.bfloat16
    x = (jnp.dot(lo, w0a1_ref[...].astype(bf),
                 preferred_element_type=jnp.float32)
         + jnp.dot(hi, w0a2_ref[...].astype(bf),
                   preferred_element_type=jnp.float32)
         + jnp.dot(f1, w0b_ref[...].astype(bf),
                   preferred_element_type=jnp.float32))
    x = jnp.maximum(x + b0_ref[...], 0.0) * s0_ref[...] + t0_ref[...]
    y = jnp.dot(x.astype(bf), w1_ref[...].astype(bf),
                preferred_element_type=jnp.float32)
    y = jnp.maximum(y + b1_ref[...], 0.0) * s1_ref[...] + t1_ref[...]
    out_ref[...] = y


def _mlp_call(interp_w, feat1f, w0a1, w0a2, w0b, w1,
              b0, s0, t0, b1, s1, t1, t2=512):
    m, cw = interp_w.shape
    c1 = feat1f.shape[1]
    h0 = w0b.shape[1]
    h1 = w1.shape[1]
    grid = (m // t2,)
    full = lambda r, c: pl.BlockSpec((r, c), lambda i: (0, 0))
    return pl.pallas_call(
        _mlp_body,
        grid=grid,
        in_specs=[
            pl.BlockSpec((t2, cw), lambda i: (i, 0)),
            pl.BlockSpec((t2, c1), lambda i: (i, 0)),
            full(cw, h0), full(cw, h0), full(c1, h0), full(h0, h1),
            full(1, h0), full(1, h0), full(1, h0),
            full(1, h1), full(1, h1), full(1, h1),
        ],
        out_specs=pl.BlockSpec((t2, h1), lambda i: (i, 0)),
        out_shape=jax.ShapeDtypeStruct((m, h1), jnp.float32),
    )(interp_w, feat1f, w0a1, w0a2, w0b, w1, b0, s0, t0, b1, s1, t1)


def kernel(xyz1, xyz2, feat1, feat2, W0, b0, g0, beta0, m0, v0,
           W1, b1, g1, beta1, m1, v1):
    bsz, n1, _ = xyz1.shape
    n2 = xyz2.shape[1]
    c1 = feat1.shape[2]
    c2 = feat2.shape[2]

    xyz2t = jnp.swapaxes(xyz2, 1, 2)
    s0 = g0 / jnp.sqrt(v0 + EPS_BN)
    t0 = beta0 - m0 * s0
    s1 = g1 / jnp.sqrt(v1 + EPS_BN)
    t1 = beta1 - m1 * s1
    hw = c2 // 2

    m = bsz * n1
    feat2w = _pack_feat2_call(feat2.reshape(bsz * n2, c2))
    iw = _top3_call(xyz1, xyz2t)
    interp_w = _sc_interpolate(iw.reshape(6, m), feat2w)
    out = _mlp_call(interp_w, feat1.reshape(m, c1),
                    W0[:hw], W0[hw:c2], W0[c2:], W1,
                    b0[None], s0[None], t0[None],
                    b1[None], s1[None], t1[None])
    return out.reshape(bsz, n1, W1.shape[1])


# speedup vs baseline: 1.2232x; 1.2232x over previous
import functools

import jax
import jax.numpy as jnp
from jax import lax
from jax.experimental import pallas as pl
from jax.experimental.pallas import tpu as pltpu
from jax.experimental.pallas import tpu_sc as plsc

EPS_BN = 1e-3
LANES = 16


def _top3_body(xyz1_ref, xyz2t_ref, iw_ref, *, n2, nb_sc):
    b = pl.program_id(0)
    x1 = xyz1_ref[0]
    x2t = xyz2t_ref[0]
    ab = lax.dot_general(x1, x2t, (((1,), (0,)), ((), ())),
                         preferred_element_type=jnp.float32)
    a2 = jnp.sum(x1 * x1, axis=1, keepdims=True)
    b2 = jnp.sum(x2t * x2t, axis=0, keepdims=True)
    d2 = jnp.clip(a2 - 2.0 * ab + b2, 0.0, 1e37)

    t1 = d2.shape[0]
    iota = lax.broadcasted_iota(jnp.int32, (t1, n2), 1)
    bias = jnp.int32(1 << 23)
    keyi = ((lax.bitcast_convert_type(d2, jnp.int32) & ~(n2 - 1)) | iota)
    keyf = lax.bitcast_convert_type(keyi + bias, jnp.float32)
    inf = jnp.float32(jnp.inf)
    ms, idxs = [], []
    for _ in range(3):
        kmf = jnp.min(keyf, axis=1, keepdims=True)
        keyf = jnp.where(keyf == kmf, inf, keyf)
        kmi = lax.bitcast_convert_type(kmf, jnp.int32) - bias
        idxs.append(kmi & (n2 - 1))
        ms.append(lax.bitcast_convert_type(kmi & ~(n2 - 1), jnp.float32))

    inv = [1.0 / (m + 1e-10) for m in ms]
    s = inv[0] + inv[1] + inv[2]
    ws = [v / s for v in inv]

    base = (b % nb_sc) * n2
    for k in range(3):
        iw_ref[k, 0] = idxs[k] + base
        iw_ref[3 + k, 0] = lax.bitcast_convert_type(ws[k], jnp.int32)


def _top3_call(xyz1, xyz2t, t1=512):
    bsz, n1, _ = xyz1.shape
    n2 = xyz2t.shape[2]
    grid = (bsz, n1 // t1)
    return pl.pallas_call(
        functools.partial(_top3_body, n2=n2, nb_sc=bsz // 2),
        grid=grid,
        in_specs=[
            pl.BlockSpec((1, t1, 3), lambda b, i: (b, i, 0)),
            pl.BlockSpec((1, 3, n2), lambda b, i: (b, 0, 0)),
        ],
        out_specs=pl.BlockSpec((6, 1, t1, 1), lambda b, i: (0, b, i, 0)),
        out_shape=jax.ShapeDtypeStruct((6, bsz, n1, 1), jnp.int32),
    )(xyz1, xyz2t)


def _sc_interpolate(iw, feat2f, *, q=32):
    m, cw = iw.shape[1], feat2f.shape[1]
    f2rows = feat2f.shape[0]
    info = plsc.get_sparse_core_info()
    nw = info.num_cores * info.num_subcores
    nq = m // nw
    nchunk = nq // q
    mesh = plsc.VectorSubcoreMesh(core_axis_name="c", subcore_axis_name="s")
    rows = lambda: pltpu.VMEM((q, cw), jnp.int32)

    @functools.partial(
        pl.kernel, mesh=mesh,
        out_type=jax.ShapeDtypeStruct((m, cw), jnp.int32),
        scratch_types=[
            pltpu.VMEM((nq,), jnp.int32),
            pltpu.VMEM((nq,), jnp.int32),
            pltpu.VMEM((nq,), jnp.int32),
            pltpu.VMEM((nq,), jnp.int32),
            pltpu.VMEM((nq,), jnp.int32),
            pltpu.VMEM((nq,), jnp.int32),
            rows(), rows(), rows(), rows(), rows(), rows(),
            rows(), rows(),
            pltpu.VMEM_SHARED((f2rows // 2, cw), jnp.int32),
            pltpu.SemaphoreType.DMA, pltpu.SemaphoreType.DMA,
            pltpu.SemaphoreType.DMA, pltpu.SemaphoreType.DMA,
        ],
    )
    def body(iw_h, f2_h, out_h,
             i0_v, i1_v, i2_v, w0_v, w1_v, w2_v,
             r00, r10, r20, r01, r11, r21, o0, o1, f2s_v,
             gs0, gs1, os0, os1):
        cc = lax.axis_index("c")
        sid = lax.axis_index("s")
        wid = cc * info.num_subcores + sid
        base0 = wid * nq
        sla = pl.ds(base0, nq)

        @pl.when(sid == 0)
        def _stage():
            pltpu.sync_copy(f2_h.at[pl.ds(cc * (f2rows // 2), f2rows // 2)],
                            f2s_v)

        for k, dst in enumerate((i0_v, i1_v, i2_v, w0_v, w1_v, w2_v)):
            pltpu.sync_copy(iw_h.at[k, sla], dst)
        plsc.subcore_barrier()

        rbufs = ((r00, r10, r20), (r01, r11, r21))
        obufs = (o0, o1)
        gsems = (gs0, gs1)
        osems = (os0, os1)
        ivs = (i0_v, i1_v, i2_v)
        wvs = (w0_v, w1_v, w2_v)

        def issue(c, p):
            lsl = pl.ds(c * q, q)
            for k in range(3):
                pltpu.async_copy(f2s_v.at[ivs[k].at[lsl]], rbufs[p][k],
                                 gsems[p])

        issue(0, 0)

        def pair(cp, _):
            for p in (0, 1):
                c = cp * 2 + p

                @pl.when(c + 1 < nchunk)
                def _prefetch():
                    issue(c + 1, 1 - p)

                for k in range(3):
                    pltpu.make_async_copy(
                        f2s_v.at[ivs[k].at[pl.ds(0, q)]], rbufs[p][k],
                        gsems[p]).wait()

                @pl.when(c >= 2)
                def _drain_out():
                    pltpu.make_async_copy(
                        obufs[p], out_h.at[pl.ds(base0, q)], osems[p]).wait()

                wvecs = [[lax.bitcast_convert_type(
                              wv[pl.ds(c * q + h * LANES, LANES)], jnp.float32)
                          for h in range(q // LANES)] for wv in wvs]
                dnums = lax.GatherDimensionNumbers(
                    offset_dims=(), collapsed_slice_dims=(0,),
                    start_index_map=(0,))
                bcast = lambda v, lane: lax.gather(
                    v, lane[:, None], dnums, (1,),
                    mode=lax.GatherScatterMode.PROMISE_IN_BOUNDS)
                for qq in range(q):
                    lane = jnp.full((LANES,), qq % LANES, jnp.int32)
                    h = qq // LANES
                    wa = bcast(wvecs[0][h], lane)
                    wb = bcast(wvecs[1][h], lane)
                    wc = bcast(wvecs[2][h], lane)
                    himask = jnp.int32(-65536)
                    rnd = jnp.int32(0x8000)
                    for j in range(cw // LANES):
                        ds = pl.ds(j * LANES, LANES)
                        ai, bi = [], []
                        for k in range(3):
                            xi = rbufs[p][k][qq, ds]
                            ai.append(lax.bitcast_convert_type(xi << 16, jnp.float32))
                            bi.append(lax.bitcast_convert_type(xi & himask, jnp.float32))
                        oa = ai[0] * wa + ai[1] * wb + ai[2] * wc
                        ob = bi[0] * wa + bi[1] * wb + bi[2] * wc
                        oi = lax.shift_right_logical(
                            lax.bitcast_convert_type(oa, jnp.int32) + rnd, 16) | (
                            (lax.bitcast_convert_type(ob, jnp.int32) + rnd) & himask)
                        obufs[p][qq, ds] = oi
                pltpu.async_copy(obufs[p], out_h.at[pl.ds(base0 + c * q, q)],
                                 osems[p])
            return 0

        lax.fori_loop(0, nchunk // 2, pair, 0)
        for p in (0, 1):
            pltpu.make_async_copy(obufs[p], out_h.at[pl.ds(base0, q)],
                                  osems[p]).wait()

    return body(iw, feat2f)


def _pack_body(x_ref, out_ref):
    xi = lax.bitcast_convert_type(x_ref[...], jnp.int32)
    r = xi + 0x7FFF + (lax.shift_right_logical(xi, 16) & 1)
    n = out_ref.shape[1]
    lo = lax.shift_right_logical(r[:, :n], 16)
    hi = r[:, n:] & jnp.int32(-65536)
    out_ref[...] = lo | hi


def _pack_feat2_call(feat2f, blk=1024):
    n, c2 = feat2f.shape
    return pl.pallas_call(
        _pack_body,
        grid=(n // blk,),
        in_specs=[pl.BlockSpec((blk, c2), lambda i: (i, 0))],
        out_specs=pl.BlockSpec((blk, c2 // 2), lambda i: (i, 0)),
        out_shape=jax.ShapeDtypeStruct((n, c2 // 2), jnp.int32),
    )(feat2f)


def _mlp_body(interp_ref, feat1_ref, w0a1_ref, w0a2_ref, w0b_ref, w1_ref,
              b0_ref, s0_ref, t0_ref, b1_ref, s1_ref, t1_ref, out_ref):
    wi = interp_ref[...]
    lo = lax.bitcast_convert_type(wi << 16, jnp.float32).astype(jnp.bfloat16)
    hi = lax.bitcast_convert_type(wi & jnp.int32(-65536),
                                  jnp.float32).astype(jnp.bfloat16)
    f1 = feat1_ref[...].astype(jnp.bfloat16)
    bf = jnp.bfloat16
    x = (jnp.dot(lo, w0a1_ref[...].astype(bf),
                 preferred_element_type=jnp.float32)
         + jnp.dot(hi, w0a2_ref[...].astype(bf),
                   preferred_element_type=jnp.float32)
         + jnp.dot(f1, w0b_ref[...].astype(bf),
                   preferred_element_type=jnp.float32))
    x = jnp.maximum(x + b0_ref[...], 0.0) * s0_ref[...] + t0_ref[...]
    y = jnp.dot(x.astype(bf), w1_ref[...].astype(bf),
                preferred_element_type=jnp.float32)
    y = jnp.maximum(y + b1_ref[...], 0.0) * s1_ref[...] + t1_ref[...]
    out_ref[...] = y


def _mlp_call(interp_w, feat1f, w0a1, w0a2, w0b, w1,
              b0, s0, t0, b1, s1, t1, t2=512):
    m, cw = interp_w.shape
    c1 = feat1f.shape[1]
    h0 = w0b.shape[1]
    h1 = w1.shape[1]
    grid = (m // t2,)
    full = lambda r, c: pl.BlockSpec((r, c), lambda i: (0, 0))
    return pl.pallas_call(
        _mlp_body,
        grid=grid,
        in_specs=[
            pl.BlockSpec((t2, cw), lambda i: (i, 0)),
            pl.BlockSpec((t2, c1), lambda i: (i, 0)),
            full(cw, h0), full(cw, h0), full(c1, h0), full(h0, h1),
            full(1, h0), full(1, h0), full(1, h0),
            full(1, h1), full(1, h1), full(1, h1),
        ],
        out_specs=pl.BlockSpec((t2, h1), lambda i: (i, 0)),
        out_shape=jax.ShapeDtypeStruct((m, h1), jnp.float32),
    )(interp_w, feat1f, w0a1, w0a2, w0b, w1, b0, s0, t0, b1, s1, t1)


def kernel(xyz1, xyz2, feat1, feat2, W0, b0, g0, beta0, m0, v0,
           W1, b1, g1, beta1, m1, v1):
    bsz, n1, _ = xyz1.shape
    n2 = xyz2.shape[1]
    c1 = feat1.shape[2]
    c2 = feat2.shape[2]

    xyz2t = jnp.swapaxes(xyz2, 1, 2)
    s0 = g0 / jnp.sqrt(v0 + EPS_BN)
    t0 = beta0 - m0 * s0
    s1 = g1 / jnp.sqrt(v1 + EPS_BN)
    t1 = beta1 - m1 * s1
    hw = c2 // 2

    hb = bsz // 2
    mh = hb * n1
    feat2w = _pack_feat2_call(feat2.reshape(bsz * n2, c2))
    interps, f1s = [], []
    for h in range(2):
        bs = slice(h * hb, (h + 1) * hb)
        iw = _top3_call(xyz1[bs], xyz2t[bs])
        interp_w = _sc_interpolate(
            iw.reshape(6, mh), feat2w[h * hb * n2:(h + 1) * hb * n2])
        interps.append(interp_w)
        f1s.append(feat1[bs].reshape(mh, c1))

    outs = [_mlp_call(interp_w, f1, W0[:hw], W0[hw:c2], W0[c2:], W1,
                      b0[None], s0[None], t0[None],
                      b1[None], s1[None], t1[None])
            for interp_w, f1 in zip(interps, f1s)]
    out = jnp.concatenate(outs, axis=0)
    return out.reshape(bsz, n1, W1.shape[1])


# speedup vs baseline: 1.2640x; 1.0333x over previous
import functools

import jax
import jax.numpy as jnp
from jax import lax
from jax.experimental import pallas as pl
from jax.experimental.pallas import tpu as pltpu
from jax.experimental.pallas import tpu_sc as plsc

EPS_BN = 1e-3
LANES = 16


def _top3_body(xyz1_ref, xyz2t_ref, iw_ref, *, n2, nb_sc):
    b = pl.program_id(0)
    x1 = xyz1_ref[0]
    x2t = xyz2t_ref[0]
    ab = lax.dot_general(x1, x2t, (((1,), (0,)), ((), ())),
                         preferred_element_type=jnp.float32)
    a2 = jnp.sum(x1 * x1, axis=1, keepdims=True)
    b2 = jnp.sum(x2t * x2t, axis=0, keepdims=True)
    d2 = jnp.clip(a2 - 2.0 * ab + b2, 0.0, 1e37)

    t1 = d2.shape[0]
    iota = lax.broadcasted_iota(jnp.int32, (t1, n2), 1)
    bias = jnp.int32(1 << 23)
    keyi = ((lax.bitcast_convert_type(d2, jnp.int32) & ~(n2 - 1)) | iota)
    keyf = lax.bitcast_convert_type(keyi + bias, jnp.float32)
    inf = jnp.float32(jnp.inf)
    ms, idxs = [], []
    for _ in range(3):
        kmf = jnp.min(keyf, axis=1, keepdims=True)
        keyf = jnp.where(keyf == kmf, inf, keyf)
        kmi = lax.bitcast_convert_type(kmf, jnp.int32) - bias
        idxs.append(kmi & (n2 - 1))
        ms.append(lax.bitcast_convert_type(kmi & ~(n2 - 1), jnp.float32))

    inv = [1.0 / (m + 1e-10) for m in ms]
    s = inv[0] + inv[1] + inv[2]
    ws = [v / s for v in inv]

    base = (b % nb_sc) * n2
    for k in range(3):
        iw_ref[k, 0] = idxs[k] + base
        iw_ref[3 + k, 0] = lax.bitcast_convert_type(ws[k], jnp.int32)


def _top3_call(xyz1, xyz2t, t1=512):
    bsz, n1, _ = xyz1.shape
    n2 = xyz2t.shape[2]
    grid = (bsz, n1 // t1)
    return pl.pallas_call(
        functools.partial(_top3_body, n2=n2, nb_sc=bsz // 2),
        grid=grid,
        in_specs=[
            pl.BlockSpec((1, t1, 3), lambda b, i: (b, i, 0)),
            pl.BlockSpec((1, 3, n2), lambda b, i: (b, 0, 0)),
        ],
        out_specs=pl.BlockSpec((6, 1, t1, 1), lambda b, i: (0, b, i, 0)),
        out_shape=jax.ShapeDtypeStruct((6, bsz, n1, 1), jnp.int32),
    )(xyz1, xyz2t)


def _sc_interpolate(iw, feat2f, *, q=32):
    m, cw = iw.shape[1], feat2f.shape[1]
    f2rows = feat2f.shape[0]
    info = plsc.get_sparse_core_info()
    nw = info.num_cores * info.num_subcores
    nq = m // nw
    nchunk = nq // q
    mesh = plsc.VectorSubcoreMesh(core_axis_name="c", subcore_axis_name="s")
    rows = lambda: pltpu.VMEM((q, cw), jnp.int32)

    @functools.partial(
        pl.kernel, mesh=mesh,
        out_type=jax.ShapeDtypeStruct((m, cw), jnp.int32),
        scratch_types=[
            pltpu.VMEM((nq,), jnp.int32),
            pltpu.VMEM((nq,), jnp.int32),
            pltpu.VMEM((nq,), jnp.int32),
            pltpu.VMEM((nq,), jnp.int32),
            pltpu.VMEM((nq,), jnp.int32),
            pltpu.VMEM((nq,), jnp.int32),
            rows(), rows(), rows(), rows(), rows(), rows(),
            rows(), rows(),
            pltpu.VMEM_SHARED((f2rows // 2, cw), jnp.int32),
            pltpu.SemaphoreType.DMA, pltpu.SemaphoreType.DMA,
            pltpu.SemaphoreType.DMA, pltpu.SemaphoreType.DMA,
        ],
    )
    def body(iw_h, f2_h, out_h,
             i0_v, i1_v, i2_v, w0_v, w1_v, w2_v,
             r00, r10, r20, r01, r11, r21, o0, o1, f2s_v,
             gs0, gs1, os0, os1):
        cc = lax.axis_index("c")
        sid = lax.axis_index("s")
        wid = cc * info.num_subcores + sid
        base0 = wid * nq
        sla = pl.ds(base0, nq)

        @pl.when(sid == 0)
        def _stage():
            pltpu.sync_copy(f2_h.at[pl.ds(cc * (f2rows // 2), f2rows // 2)],
                            f2s_v)

        for k, dst in enumerate((i0_v, i1_v, i2_v, w0_v, w1_v, w2_v)):
            pltpu.sync_copy(iw_h.at[k, sla], dst)
        plsc.subcore_barrier()

        rbufs = ((r00, r10, r20), (r01, r11, r21))
        obufs = (o0, o1)
        gsems = (gs0, gs1)
        osems = (os0, os1)
        ivs = (i0_v, i1_v, i2_v)
        wvs = (w0_v, w1_v, w2_v)

        def issue(c, p):
            lsl = pl.ds(c * q, q)
            for k in range(3):
                pltpu.async_copy(f2s_v.at[ivs[k].at[lsl]], rbufs[p][k],
                                 gsems[p])

        issue(0, 0)

        def pair(cp, _):
            for p in (0, 1):
                c = cp * 2 + p

                @pl.when(c + 1 < nchunk)
                def _prefetch():
                    issue(c + 1, 1 - p)

                for k in range(3):
                    pltpu.make_async_copy(
                        f2s_v.at[ivs[k].at[pl.ds(0, q)]], rbufs[p][k],
                        gsems[p]).wait()

                @pl.when(c >= 2)
                def _drain_out():
                    pltpu.make_async_copy(
                        obufs[p], out_h.at[pl.ds(base0, q)], osems[p]).wait()

                wvecs = [[lax.bitcast_convert_type(
                              wv[pl.ds(c * q + h * LANES, LANES)], jnp.float32)
                          for h in range(q // LANES)] for wv in wvs]
                dnums = lax.GatherDimensionNumbers(
                    offset_dims=(), collapsed_slice_dims=(0,),
                    start_index_map=(0,))
                bcast = lambda v, lane: lax.gather(
                    v, lane[:, None], dnums, (1,),
                    mode=lax.GatherScatterMode.PROMISE_IN_BOUNDS)
                for qq in range(q):
                    lane = jnp.full((LANES,), qq % LANES, jnp.int32)
                    h = qq // LANES
                    wa = bcast(wvecs[0][h], lane)
                    wb = bcast(wvecs[1][h], lane)
                    wc = bcast(wvecs[2][h], lane)
                    himask = jnp.int32(-65536)
                    rnd = jnp.int32(0x8000)
                    for j in range(cw // LANES):
                        ds = pl.ds(j * LANES, LANES)
                        ai, bi = [], []
                        for k in range(3):
                            xi = rbufs[p][k][qq, ds]
                            ai.append(lax.bitcast_convert_type(xi << 16, jnp.float32))
                            bi.append(lax.bitcast_convert_type(xi & himask, jnp.float32))
                        oa = ai[0] * wa + ai[1] * wb + ai[2] * wc
                        ob = bi[0] * wa + bi[1] * wb + bi[2] * wc
                        oi = lax.shift_right_logical(
                            lax.bitcast_convert_type(oa, jnp.int32) + rnd, 16) | (
                            (lax.bitcast_convert_type(ob, jnp.int32) + rnd) & himask)
                        obufs[p][qq, ds] = oi
                pltpu.async_copy(obufs[p], out_h.at[pl.ds(base0 + c * q, q)],
                                 osems[p])
            return 0

        lax.fori_loop(0, nchunk // 2, pair, 0)
        for p in (0, 1):
            pltpu.make_async_copy(obufs[p], out_h.at[pl.ds(base0, q)],
                                  osems[p]).wait()

    return body(iw, feat2f)


def _pack_body(x_ref, out_ref):
    xi = lax.bitcast_convert_type(x_ref[...], jnp.int32)
    r = xi + 0x7FFF + (lax.shift_right_logical(xi, 16) & 1)
    n = out_ref.shape[1]
    lo = lax.shift_right_logical(r[:, :n], 16)
    hi = r[:, n:] & jnp.int32(-65536)
    out_ref[...] = lo | hi


def _pack_feat2_call(feat2f, blk=1024):
    n, c2 = feat2f.shape
    return pl.pallas_call(
        _pack_body,
        grid=(n // blk,),
        in_specs=[pl.BlockSpec((blk, c2), lambda i: (i, 0))],
        out_specs=pl.BlockSpec((blk, c2 // 2), lambda i: (i, 0)),
        out_shape=jax.ShapeDtypeStruct((n, c2 // 2), jnp.int32),
    )(feat2f)


def _mlp_body(interp_ref, feat1_ref, w0a1_ref, w0a2_ref, w0b_ref, w1_ref,
              b0_ref, s0_ref, t0_ref, b1_ref, s1_ref, t1_ref, out_ref):
    wi = interp_ref[...]
    lo = lax.bitcast_convert_type(wi << 16, jnp.float32).astype(jnp.bfloat16)
    hi = lax.bitcast_convert_type(wi & jnp.int32(-65536),
                                  jnp.float32).astype(jnp.bfloat16)
    f1 = feat1_ref[...].astype(jnp.bfloat16)
    bf = jnp.bfloat16
    x = (jnp.dot(lo, w0a1_ref[...].astype(bf),
                 preferred_element_type=jnp.float32)
         + jnp.dot(hi, w0a2_ref[...].astype(bf),
                   preferred_element_type=jnp.float32)
         + jnp.dot(f1, w0b_ref[...].astype(bf),
                   preferred_element_type=jnp.float32))
    x = jnp.maximum(x + b0_ref[...], 0.0) * s0_ref[...] + t0_ref[...]
    y = jnp.dot(x.astype(bf), w1_ref[...].astype(bf),
                preferred_element_type=jnp.float32)
    y = jnp.maximum(y + b1_ref[...], 0.0) * s1_ref[...] + t1_ref[...]
    out_ref[...] = y


def _mlp_call(interp_w, feat1f, w0a1, w0a2, w0b, w1,
              b0, s0, t0, b1, s1, t1, t2=512):
    m, cw = interp_w.shape
    c1 = feat1f.shape[1]
    h0 = w0b.shape[1]
    h1 = w1.shape[1]
    grid = (m // t2,)
    full = lambda r, c: pl.BlockSpec((r, c), lambda i: (0, 0))
    return pl.pallas_call(
        _mlp_body,
        grid=grid,
        in_specs=[
            pl.BlockSpec((t2, cw), lambda i: (i, 0)),
            pl.BlockSpec((t2, c1), lambda i: (i, 0)),
            full(cw, h0), full(cw, h0), full(c1, h0), full(h0, h1),
            full(1, h0), full(1, h0), full(1, h0),
            full(1, h1), full(1, h1), full(1, h1),
        ],
        out_specs=pl.BlockSpec((t2, h1), lambda i: (i, 0)),
        out_shape=jax.ShapeDtypeStruct((m, h1), jnp.float32),
    )(interp_w, feat1f, w0a1, w0a2, w0b, w1, b0, s0, t0, b1, s1, t1)


def kernel(xyz1, xyz2, feat1, feat2, W0, b0, g0, beta0, m0, v0,
           W1, b1, g1, beta1, m1, v1):
    bsz, n1, _ = xyz1.shape
    n2 = xyz2.shape[1]
    c1 = feat1.shape[2]
    c2 = feat2.shape[2]

    xyz2t = jnp.swapaxes(xyz2, 1, 2)
    s0 = g0 / jnp.sqrt(v0 + EPS_BN)
    t0 = beta0 - m0 * s0
    s1 = g1 / jnp.sqrt(v1 + EPS_BN)
    t1 = beta1 - m1 * s1
    hw = c2 // 2

    hb = bsz // 4
    mh = hb * n1
    feat2w = _pack_feat2_call(feat2.reshape(bsz * n2, c2))
    interps, f1s = [], []
    for h in range(4):
        bs = slice(h * hb, (h + 1) * hb)
        iw = _top3_call(xyz1[bs], xyz2t[bs])
        interp_w = _sc_interpolate(
            iw.reshape(6, mh), feat2w[h * hb * n2:(h + 1) * hb * n2])
        interps.append(interp_w)
        f1s.append(feat1[bs].reshape(mh, c1))

    outs = [_mlp_call(interp_w, f1, W0[:hw], W0[hw:c2], W0[c2:], W1,
                      b0[None], s0[None], t0[None],
                      b1[None], s1[None], t1[None])
            for interp_w, f1 in zip(interps, f1s)]
    out = jnp.concatenate(outs, axis=0)
    return out.reshape(bsz, n1, W1.shape[1])
